# Initial kernel scaffold; baseline (speedup 1.0000x reference)
import jax
import jax.numpy as jnp
from jax.experimental import pallas as pl


def kernel(neural_prob_mtx, features, W, b):
    raise NotImplementedError("write your pallas kernel here")


# speedup vs baseline: 1.0003x; 1.0003x over previous
import jax
import jax.numpy as jnp
from jax.experimental import pallas as pl
from jax.experimental.pallas import tpu as pltpu

_TOPK = 128
_FEAT = 3


def _score_body(candi_probs_ref, feats_ref, w_ref, b_ref, src_ref):
    f = feats_ref[...]
    w = w_ref[...]
    logits = (
        f[0] * w[0, 0] + f[1] * w[1, 0] + f[2] * w[2, 0]
        + b_ref[0]
    )
    score = jnp.exp(logits)
    l1 = jnp.maximum(jnp.sum(jnp.abs(score), axis=1, keepdims=True), 1e-12)
    candi_sum = jnp.sum(candi_probs_ref[...], axis=1, keepdims=True)
    src_ref[...] = score / l1 * candi_sum


def kernel(neural_prob_mtx, features, W, b):
    n1 = neural_prob_mtx.shape[0]
    candi_probs, candi_idxes = jax.lax.top_k(neural_prob_mtx, _TOPK)
    rblk = 256
    feats_t = jnp.transpose(features, (2, 0, 1))
    src = pl.pallas_call(
        _score_body,
        grid=(n1 // rblk,),
        in_specs=[
            pl.BlockSpec((rblk, _TOPK), lambda i: (i, 0)),
            pl.BlockSpec((_FEAT, rblk, _TOPK), lambda i: (0, i, 0)),
            pl.BlockSpec((_FEAT, 1), lambda i: (0, 0)),
            pl.BlockSpec((1,), lambda i: (0,)),
        ],
        out_specs=pl.BlockSpec((rblk, _TOPK), lambda i: (i, 0)),
        out_shape=jax.ShapeDtypeStruct((n1, _TOPK), jnp.float32),
    )(candi_probs, feats_t, W, b)
    rows = jnp.arange(n1)[:, None]
    return neural_prob_mtx.at[rows, candi_idxes].set(src)


# speedup vs baseline: 8.6984x; 8.6958x over previous
import functools

import jax
import jax.numpy as jnp
from jax import lax
from jax.experimental import pallas as pl
from jax.experimental.pallas import tpu as pltpu
from jax.experimental.pallas import tpu_sc as plsc

N1 = 4096
N2 = 16384
K = 128
NC = 2
NS = 16
L = 16
ROWS_PER_W = N1 // (NC * NS)
NSEG = 128
SEG_G = 8
SEG_V = N2 // NSEG
BIG_NEG = -3.0e38


def _scalar(x):
    if getattr(x, "shape", ()) == (L,):
        return x[0]
    return x


def _maxtree(vs):
    while len(vs) > 1:
        vs = [jnp.maximum(vs[2 * i], vs[2 * i + 1]) for i in range(len(vs) // 2)] + (
            [vs[-1]] if len(vs) % 2 else []
        )
    return vs[0]


def _body(neural_hbm, feats_hbm, wb_hbm, out_hbm,
          rowbuf, featbuf, wbbuf, scorebuf):
    wid = lax.axis_index("s") * NC + lax.axis_index("c")
    base_row = wid * ROWS_PER_W

    pltpu.sync_copy(wb_hbm, wbbuf)

    iota = lax.iota(jnp.int32, L)
    fiota = iota.astype(jnp.float32)
    bases = [iota * NSEG + (L * NSEG) * t for t in range(SEG_G)]

    def do_row(r, _):
        row = base_row + r
        pltpu.sync_copy(neural_hbm.at[row], rowbuf)
        pltpu.sync_copy(feats_hbm.at[row], featbuf)

        def seg_step(v, Ms):
            off = v * NSEG
            return tuple(
                jnp.maximum(Ms[g], rowbuf[pl.ds(off + g * L, L)])
                for g in range(SEG_G)
            )
        M = lax.fori_loop(
            0, SEG_V, seg_step,
            tuple(jnp.full((L,), BIG_NEG, jnp.float32) for _ in range(SEG_G)),
        )

        def extract(k, carry):
            M, idxv, csum = carry
            gmax = _scalar(jnp.max(_maxtree(list(M))))
            masks = [Mg == gmax for Mg in M]
            cvec = masks[0].astype(jnp.int32)
            gvec = jnp.zeros((L,), jnp.int32)
            for g in range(1, SEG_G):
                mi = masks[g].astype(jnp.int32)
                cvec = cvec + mi
                gvec = gvec + mi * g
            total = _scalar(jnp.sum(cvec))

            def common(_):
                gstar = _scalar(jnp.sum(gvec))
                lstar = _scalar(jnp.sum(jnp.where(cvec > 0, iota, 0)))
                col = gstar * L + lstar
                vcand = [
                    jnp.where(
                        plsc.load_gather(rowbuf, [bases[t] + col]) == gmax,
                        iota + (L * t), jnp.int32(99999))
                    for t in range(SEG_G)
                ]
                vstar = _scalar(jnp.min(_mintree(vcand)))
                return vstar * NSEG + col

            def rare(_):
                def scan_step(i, acc):
                    x = rowbuf[pl.ds(i * L, L)]
                    return jnp.minimum(acc, jnp.where(x == gmax, iota + i * L,
                                                      jnp.int32(0x7FFFFFF)))
                acc = lax.fori_loop(0, N2 // L, scan_step,
                                    jnp.full((L,), 0x7FFFFFF, jnp.int32))
                return _scalar(jnp.min(acc))

            e = lax.cond(total == 1, common, rare, 0)

            k_vreg = k // L
            k_lane = jnp.remainder(k, L)
            klmask = iota == k_lane
            idxv = tuple(
                jnp.where(jnp.logical_and(k_vreg == j, klmask), e, idxv[j])
                for j in range(K // L)
            )
            col_e = jnp.remainder(e, NSEG)
            lane_e = jnp.remainder(col_e, L)
            g_e = col_e // L
            onelane = iota == lane_e
            plsc.store_scatter(rowbuf, [jnp.full((L,), e, jnp.int32)],
                               jnp.full((L,), BIG_NEG, jnp.float32),
                               mask=onelane)
            seg = [plsc.load_gather(rowbuf, [bases[t] + col_e])
                   for t in range(SEG_G)]
            newmax = _scalar(jnp.max(_maxtree(seg)))
            M2 = tuple(
                jnp.where(jnp.logical_and(g_e == g, onelane), newmax, M[g])
                for g in range(SEG_G)
            )
            return M2, idxv, csum + gmax

        M, idxv, csum = lax.fori_loop(
            0, K, extract,
            (M, tuple(jnp.zeros((L,), jnp.int32) for _ in range(K // L)),
             jnp.float32(0.0)))

        wv = wbbuf[pl.ds(0, L)]
        w0 = wv[0]
        w1 = wv[1]
        w2 = wv[2]
        b0 = wv[3]
        ssum = jnp.zeros((L,), jnp.float32)
        for j in range(K // L):
            fbase = (iota + j * L) * 3
            f0 = plsc.load_gather(featbuf, [fbase])
            f1 = plsc.load_gather(featbuf, [fbase + 1])
            f2 = plsc.load_gather(featbuf, [fbase + 2])
            s = jnp.exp(f0 * w0 + f1 * w1 + f2 * w2 + b0)
            scorebuf[pl.ds(j * L, L)] = s
            ssum = ssum + s
        l1 = jnp.maximum(_scalar(jnp.sum(ssum)), jnp.float32(1e-12))
        scale = jnp.broadcast_to(csum, (L,)) / jnp.broadcast_to(l1, (L,))

        for j in range(K // L):
            src = scorebuf[pl.ds(j * L, L)] * scale
            plsc.store_scatter(rowbuf, [idxv[j]], src)
        pltpu.sync_copy(rowbuf, out_hbm.at[row])
        return 0

    lax.fori_loop(0, ROWS_PER_W, do_row, 0)


def _mintree(vs):
    while len(vs) > 1:
        vs = [jnp.minimum(vs[2 * i], vs[2 * i + 1]) for i in range(len(vs) // 2)] + (
            [vs[-1]] if len(vs) % 2 else []
        )
    return vs[0]


@jax.jit
def kernel(neural_prob_mtx, features, W, b):
    feats = features.reshape(N1, K * 3)
    wb = jnp.zeros((16,), jnp.float32)
    wb = wb.at[0].set(W[0, 0]).at[1].set(W[1, 0]).at[2].set(W[2, 0]).at[3].set(b[0])

    mesh = plsc.VectorSubcoreMesh(core_axis_name="c", subcore_axis_name="s")
    run = pl.kernel(
        _body,
        out_type=jax.ShapeDtypeStruct((N1, N2), jnp.float32),
        mesh=mesh,
        scratch_types=[
            pltpu.VMEM((N2,), jnp.float32),
            pltpu.VMEM((K * 3,), jnp.float32),
            pltpu.VMEM((16,), jnp.float32),
            pltpu.VMEM((K,), jnp.float32),
        ],
        compiler_params=pltpu.CompilerParams(needs_layout_passes=False),
    )
    return run(neural_prob_mtx, feats, wb)


# speedup vs baseline: 11.8679x; 1.3644x over previous
import functools

import jax
import jax.numpy as jnp
from jax import lax
from jax.experimental import pallas as pl
from jax.experimental.pallas import tpu as pltpu
from jax.experimental.pallas import tpu_sc as plsc

N1 = 4096
N2 = 16384
K = 128
NC = 2
NS = 16
L = 16
ROWS_PER_W = N1 // (NC * NS)
NSEG = 128
SEG_G = 8
SEG_V = N2 // NSEG
BIG_NEG = -3.0e38


def _scalar(x):
    if getattr(x, "shape", ()) == (L,):
        return x[0]
    return x


def _maxtree(vs):
    while len(vs) > 1:
        vs = [jnp.maximum(vs[2 * i], vs[2 * i + 1]) for i in range(len(vs) // 2)] + (
            [vs[-1]] if len(vs) % 2 else []
        )
    return vs[0]


def _body(neural_hbm, feats_hbm, wb_hbm, out_hbm,
          rowbuf, featbuf, wbbuf, scorebuf):
    wid = lax.axis_index("s") * NC + lax.axis_index("c")
    base_row = wid * ROWS_PER_W

    pltpu.sync_copy(wb_hbm, wbbuf)

    iota = lax.iota(jnp.int32, L)
    fiota = iota.astype(jnp.float32)
    bases = [iota * NSEG + (L * NSEG) * t for t in range(SEG_G)]

    def do_row(r, _):
        row = base_row + r
        pltpu.sync_copy(neural_hbm.at[row], rowbuf)
        pltpu.sync_copy(feats_hbm.at[row], featbuf)

        def seg_step(v4, Ms):
            off = v4 * (NSEG * 4)
            for u in range(4):
                Ms = tuple(
                    jnp.maximum(Ms[g], rowbuf[pl.ds(off + u * NSEG + g * L, L)])
                    for g in range(SEG_G)
                )
            return Ms
        M = lax.fori_loop(
            0, SEG_V // 4, seg_step,
            tuple(jnp.full((L,), BIG_NEG, jnp.float32) for _ in range(SEG_G)),
        )

        def extract(k, carry):
            M, idxv, csum = carry
            gmax = _scalar(jnp.max(_maxtree(list(M))))
            masks = [Mg == gmax for Mg in M]
            pcs = [_scalar(plsc.all_reduce_population_count(m)) for m in masks]
            total = pcs[0]
            gweight = pcs[1]
            for g in range(2, SEG_G):
                total = total + pcs[g]
                gweight = gweight + pcs[g] * g
            total = total + pcs[1]
            orm = masks[0]
            for g in range(1, SEG_G):
                orm = jnp.logical_or(orm, masks[g])

            def patch_seg(seg, t_e, lane_e):
                lmask = iota == lane_e
                return tuple(
                    jnp.where(jnp.logical_and(t_e == t, lmask),
                              jnp.float32(BIG_NEG), seg[t])
                    for t in range(SEG_G)
                )

            def common(_):
                lstar = _scalar(plsc.all_reduce_ffs(orm))
                col = gweight * L + lstar
                seg = [plsc.load_gather(rowbuf, [bases[t] + col])
                       for t in range(SEG_G)]
                vcand = [
                    jnp.where(seg[t] == gmax, iota + (L * t), jnp.int32(99999))
                    for t in range(SEG_G)
                ]
                vstar = _scalar(jnp.min(_mintree(vcand)))
                segp = patch_seg(seg, vstar // L, jnp.remainder(vstar, L))
                return (vstar * NSEG + col,) + segp

            def rare(_):
                def scan_step(i, acc):
                    x = rowbuf[pl.ds(i * L, L)]
                    return jnp.minimum(acc, jnp.where(x == gmax, iota + i * L,
                                                      jnp.int32(0x7FFFFFF)))
                acc = lax.fori_loop(0, N2 // L, scan_step,
                                    jnp.full((L,), 0x7FFFFFF, jnp.int32))
                e = _scalar(jnp.min(acc))
                col_e = jnp.remainder(e, NSEG)
                v_e = e // NSEG
                seg = [plsc.load_gather(rowbuf, [bases[t] + col_e])
                       for t in range(SEG_G)]
                segp = patch_seg(seg, v_e // L, jnp.remainder(v_e, L))
                return (e,) + segp

            res = lax.cond(total == 1, common, rare, 0)
            e = res[0]
            segp = list(res[1:])

            k_vreg = k // L
            k_lane = jnp.remainder(k, L)
            klmask = iota == k_lane
            idxv = tuple(
                jnp.where(jnp.logical_and(k_vreg == j, klmask), e, idxv[j])
                for j in range(K // L)
            )
            col_e = jnp.remainder(e, NSEG)
            lane_e = jnp.remainder(col_e, L)
            g_e = col_e // L
            onelane = iota == lane_e
            plsc.store_scatter(rowbuf, [jnp.full((L,), e, jnp.int32)],
                               jnp.full((L,), BIG_NEG, jnp.float32),
                               mask=onelane)
            newmax = _scalar(jnp.max(_maxtree(segp)))
            M2 = tuple(
                jnp.where(jnp.logical_and(g_e == g, onelane), newmax, M[g])
                for g in range(SEG_G)
            )
            return M2, idxv, csum + gmax

        M, idxv, csum = lax.fori_loop(
            0, K, extract,
            (M, tuple(jnp.zeros((L,), jnp.int32) for _ in range(K // L)),
             jnp.float32(0.0)))

        wv = wbbuf[pl.ds(0, L)]
        w0 = wv[0]
        w1 = wv[1]
        w2 = wv[2]
        b0 = wv[3]
        ssum = jnp.zeros((L,), jnp.float32)
        for j in range(K // L):
            fbase = (iota + j * L) * 3
            f0 = plsc.load_gather(featbuf, [fbase])
            f1 = plsc.load_gather(featbuf, [fbase + 1])
            f2 = plsc.load_gather(featbuf, [fbase + 2])
            s = jnp.exp(f0 * w0 + f1 * w1 + f2 * w2 + b0)
            scorebuf[pl.ds(j * L, L)] = s
            ssum = ssum + s
        l1 = jnp.maximum(_scalar(jnp.sum(ssum)), jnp.float32(1e-12))
        scale = jnp.broadcast_to(csum, (L,)) / jnp.broadcast_to(l1, (L,))

        for j in range(K // L):
            src = scorebuf[pl.ds(j * L, L)] * scale
            plsc.store_scatter(rowbuf, [idxv[j]], src)
        pltpu.sync_copy(rowbuf, out_hbm.at[row])
        return 0

    lax.fori_loop(0, ROWS_PER_W, do_row, 0)


def _mintree(vs):
    while len(vs) > 1:
        vs = [jnp.minimum(vs[2 * i], vs[2 * i + 1]) for i in range(len(vs) // 2)] + (
            [vs[-1]] if len(vs) % 2 else []
        )
    return vs[0]


@jax.jit
def kernel(neural_prob_mtx, features, W, b):
    feats = features.reshape(N1, K * 3)
    wb = jnp.zeros((16,), jnp.float32)
    wb = wb.at[0].set(W[0, 0]).at[1].set(W[1, 0]).at[2].set(W[2, 0]).at[3].set(b[0])

    mesh = plsc.VectorSubcoreMesh(core_axis_name="c", subcore_axis_name="s")
    run = pl.kernel(
        _body,
        out_type=jax.ShapeDtypeStruct((N1, N2), jnp.float32),
        mesh=mesh,
        scratch_types=[
            pltpu.VMEM((N2,), jnp.float32),
            pltpu.VMEM((K * 3,), jnp.float32),
            pltpu.VMEM((16,), jnp.float32),
            pltpu.VMEM((K,), jnp.float32),
        ],
        compiler_params=pltpu.CompilerParams(needs_layout_passes=False),
    )
    return run(neural_prob_mtx, feats, wb)


# speedup vs baseline: 13.4668x; 1.1347x over previous
import jax
import jax.numpy as jnp
from jax import lax
from jax.experimental import pallas as pl
from jax.experimental.pallas import tpu as pltpu
from jax.experimental.pallas import tpu_sc as plsc

N1 = 4096
N2 = 16384
K = 128
NC = 2
NS = 16
L = 16
NW = NC * NS
ROWS_PER_W = N1 // NW
NSEG = 128
SEG_G = 8
SEG_V = N2 // NSEG
BIG_NEG = -3.0e38
NBUF = 4
NBODY = ROWS_PER_W // NBUF


def _scalar(x):
    if getattr(x, "shape", ()) == (L,):
        return x[0]
    return x


def _maxtree(vs):
    while len(vs) > 1:
        vs = [jnp.maximum(vs[2 * i], vs[2 * i + 1]) for i in range(len(vs) // 2)] + (
            [vs[-1]] if len(vs) % 2 else []
        )
    return vs[0]


def _mintree(vs):
    while len(vs) > 1:
        vs = [jnp.minimum(vs[2 * i], vs[2 * i + 1]) for i in range(len(vs) // 2)] + (
            [vs[-1]] if len(vs) % 2 else []
        )
    return vs[0]


def _body(neural_hbm, feats_hbm, wb_hbm, out_hbm,
          rb0, rb1, rb2, rb3, fb0, fb1, fb2, fb3, wbbuf,
          sem_in, sem_fin, sem_out):
    rowbufs = [rb0, rb1, rb2, rb3]
    featbufs = [fb0, fb1, fb2, fb3]
    wid = lax.axis_index("s") * NC + lax.axis_index("c")
    base_row = wid * ROWS_PER_W

    pltpu.sync_copy(wb_hbm, wbbuf)
    wv = wbbuf[pl.ds(0, L)]
    w0, w1, w2, b0 = wv[0], wv[1], wv[2], wv[3]

    iota = lax.iota(jnp.int32, L)
    bases = [iota * NSEG + (L * NSEG) * t for t in range(SEG_G)]

    def issue_in(b, row):
        return (
            pltpu.async_copy(neural_hbm.at[row], rowbufs[b], sem_in.at[b]),
            pltpu.async_copy(feats_hbm.at[row], featbufs[b], sem_fin.at[b]),
        )

    def wait_in(b, row):
        pltpu.make_async_copy(neural_hbm.at[row], rowbufs[b],
                              sem_in.at[b]).wait()
        pltpu.make_async_copy(feats_hbm.at[row], featbufs[b],
                              sem_fin.at[b]).wait()

    def issue_out(b, row):
        return pltpu.async_copy(rowbufs[b], out_hbm.at[row], sem_out.at[b])

    def wait_out(b, row):
        pltpu.make_async_copy(rowbufs[b], out_hbm.at[row],
                              sem_out.at[b]).wait()

    def compute_pair(bufs, fbufs, rows):
        NR = len(bufs)

        def seg_step(v4, Ms):
            off = v4 * (NSEG * 4)
            for u in range(4):
                Ms = tuple(
                    tuple(
                        jnp.maximum(Ms[s][g],
                                    bufs[s][pl.ds(off + u * NSEG + g * L, L)])
                        for g in range(SEG_G)
                    )
                    for s in range(NR)
                )
            return Ms
        M = lax.fori_loop(
            0, SEG_V // 4, seg_step,
            tuple(tuple(jnp.full((L,), BIG_NEG, jnp.float32)
                        for _ in range(SEG_G)) for _ in range(NR)),
        )

        def extract(k, carry):
            M, idxv, csum = carry
            out_M, out_idxv, out_csum = [], [], []
            k_vreg = k // L
            k_lane = jnp.remainder(k, L)
            klmask = iota == k_lane
            for s in range(NR):
                buf = bufs[s]
                Ms = M[s]
                gmax = _scalar(jnp.max(_maxtree(list(Ms))))
                masks = [Mg == gmax for Mg in Ms]
                pcs = [_scalar(plsc.all_reduce_population_count(m))
                       for m in masks]
                total = pcs[0]
                gweight = pcs[1]
                for g in range(2, SEG_G):
                    total = total + pcs[g]
                    gweight = gweight + pcs[g] * g
                total = total + pcs[1]
                orm = masks[0]
                for g in range(1, SEG_G):
                    orm = jnp.logical_or(orm, masks[g])

                lstar = _scalar(plsc.all_reduce_ffs(orm))
                col = jnp.minimum(gweight * L + lstar, NSEG - 1)
                seg = [plsc.load_gather(buf, [bases[t] + col])
                       for t in range(SEG_G)]
                vcand = [
                    jnp.where(seg[t] == gmax, iota + (L * t), jnp.int32(99999))
                    for t in range(SEG_G)
                ]
                vstar = _scalar(jnp.min(_mintree(vcand)))
                lm_c = iota == jnp.remainder(vstar, L)
                t_c = vstar // L
                segp = tuple(
                    jnp.where(jnp.logical_and(t_c == t, lm_c),
                              jnp.float32(BIG_NEG), seg[t])
                    for t in range(SEG_G)
                )
                e_c = vstar * NSEG + col

                def rare(args):
                    def scan_step(i, acc):
                        x = buf[pl.ds(i * L, L)]
                        return jnp.minimum(
                            acc, jnp.where(x == gmax, iota + i * L,
                                           jnp.int32(0x7FFFFFF)))
                    acc = lax.fori_loop(0, N2 // L, scan_step,
                                        jnp.full((L,), 0x7FFFFFF, jnp.int32))
                    e = _scalar(jnp.min(acc))
                    col_e = jnp.remainder(e, NSEG)
                    v_e = e // NSEG
                    sg = [plsc.load_gather(buf, [bases[t] + col_e])
                          for t in range(SEG_G)]
                    lm = iota == jnp.remainder(v_e, L)
                    t_e = v_e // L
                    sgp = tuple(
                        jnp.where(jnp.logical_and(t_e == t, lm),
                                  jnp.float32(BIG_NEG), sg[t])
                        for t in range(SEG_G)
                    )
                    return (e,) + sgp

                res = lax.cond(total == 1, lambda args: args, rare,
                               (e_c,) + segp)
                e = res[0]
                segp = list(res[1:])

                idxv_s = tuple(
                    jnp.where(jnp.logical_and(k_vreg == j, klmask), e,
                              idxv[s][j])
                    for j in range(K // L)
                )
                col_e = jnp.remainder(e, NSEG)
                lane_e = jnp.remainder(col_e, L)
                g_e = col_e // L
                onelane = iota == lane_e
                plsc.store_scatter(buf, [jnp.full((L,), e, jnp.int32)],
                                   jnp.full((L,), BIG_NEG, jnp.float32),
                                   mask=onelane)
                newmax = _scalar(jnp.max(_maxtree(segp)))
                M2 = tuple(
                    jnp.where(jnp.logical_and(g_e == g, onelane), newmax,
                              Ms[g])
                    for g in range(SEG_G)
                )
                out_M.append(M2)
                out_idxv.append(idxv_s)
                out_csum.append(csum[s] + gmax)
            return tuple(out_M), tuple(out_idxv), tuple(out_csum)

        M, idxv, csum = lax.fori_loop(
            0, K, extract,
            (M,
             tuple(tuple(jnp.zeros((L,), jnp.int32) for _ in range(K // L))
                   for _ in range(NR)),
             tuple(jnp.float32(0.0) for _ in range(NR))))

        for s in range(NR):
            ssum = jnp.zeros((L,), jnp.float32)
            srcs = []
            for j in range(K // L):
                fbase = (iota + j * L) * 3
                f0 = plsc.load_gather(fbufs[s], [fbase])
                f1 = plsc.load_gather(fbufs[s], [fbase + 1])
                f2 = plsc.load_gather(fbufs[s], [fbase + 2])
                sc = jnp.exp(f0 * w0 + f1 * w1 + f2 * w2 + b0)
                srcs.append(sc)
                ssum = ssum + sc
            l1 = jnp.maximum(_scalar(jnp.sum(ssum)), jnp.float32(1e-12))
            scale = jnp.broadcast_to(csum[s], (L,)) / jnp.broadcast_to(l1, (L,))
            for j in range(K // L):
                plsc.store_scatter(bufs[s], [idxv[s][j]], srcs[j] * scale)

    issue_in(0, base_row + 0)
    issue_in(1, base_row + 1)

    def pipeline_body(i2, _):
        q = base_row + i2 * NBUF

        @pl.when(i2 > 0)
        def _():
            wait_out(2, q - 2)
            wait_out(3, q - 1)

        issue_in(2, q + 2)
        issue_in(3, q + 3)

        wait_in(0, q + 0)
        wait_in(1, q + 1)
        compute_pair([rowbufs[0], rowbufs[1]],
                     [featbufs[0], featbufs[1]], (q, q + 1))
        issue_out(0, q + 0)
        issue_out(1, q + 1)

        wait_in(2, q + 2)
        wait_in(3, q + 3)
        compute_pair([rowbufs[2], rowbufs[3]],
                     [featbufs[2], featbufs[3]], (q + 2, q + 3))
        issue_out(2, q + 2)
        issue_out(3, q + 3)

        wait_out(0, q + 0)
        wait_out(1, q + 1)

        @pl.when(i2 < NBODY - 1)
        def _():
            issue_in(0, q + 4)
            issue_in(1, q + 5)

        return 0

    lax.fori_loop(0, NBODY, pipeline_body, 0)
    last = base_row + (NBODY - 1) * NBUF
    wait_out(2, last + 2)
    wait_out(3, last + 3)


@jax.jit
def kernel(neural_prob_mtx, features, W, b):
    feats = features.reshape(N1, K * 3)
    wb = jnp.zeros((16,), jnp.float32)
    wb = wb.at[0].set(W[0, 0]).at[1].set(W[1, 0]).at[2].set(W[2, 0]).at[3].set(b[0])

    mesh = plsc.VectorSubcoreMesh(core_axis_name="c", subcore_axis_name="s")
    run = pl.kernel(
        _body,
        out_type=jax.ShapeDtypeStruct((N1, N2), jnp.float32),
        mesh=mesh,
        scratch_types=[
            pltpu.VMEM((N2,), jnp.float32),
            pltpu.VMEM((N2,), jnp.float32),
            pltpu.VMEM((N2,), jnp.float32),
            pltpu.VMEM((N2,), jnp.float32),
            pltpu.VMEM((K * 3,), jnp.float32),
            pltpu.VMEM((K * 3,), jnp.float32),
            pltpu.VMEM((K * 3,), jnp.float32),
            pltpu.VMEM((K * 3,), jnp.float32),
            pltpu.VMEM((16,), jnp.float32),
            pltpu.SemaphoreType.DMA((NBUF,)),
            pltpu.SemaphoreType.DMA((NBUF,)),
            pltpu.SemaphoreType.DMA((NBUF,)),
        ],
        compiler_params=pltpu.CompilerParams(needs_layout_passes=False),
    )
    return run(neural_prob_mtx, feats, wb)


# speedup vs baseline: 15.9249x; 1.1825x over previous
import jax
import jax.numpy as jnp
from jax import lax
from jax.experimental import pallas as pl
from jax.experimental.pallas import tpu as pltpu
from jax.experimental.pallas import tpu_sc as plsc

N1 = 4096
N2 = 16384
K = 128
NC = 2
NS = 16
L = 16
NW = NC * NS
ROWS_PER_W = N1 // NW
NSEG = 128
SEG_G = 8
SEG_V = N2 // NSEG
BIG_NEG = -3.0e38
NBUF = 4
NBODY = ROWS_PER_W // NBUF


def _scalar(x):
    if getattr(x, "shape", ()) == (L,):
        return x[0]
    return x


def _maxtree(vs):
    while len(vs) > 1:
        vs = [jnp.maximum(vs[2 * i], vs[2 * i + 1]) for i in range(len(vs) // 2)] + (
            [vs[-1]] if len(vs) % 2 else []
        )
    return vs[0]


def _mintree(vs):
    while len(vs) > 1:
        vs = [jnp.minimum(vs[2 * i], vs[2 * i + 1]) for i in range(len(vs) // 2)] + (
            [vs[-1]] if len(vs) % 2 else []
        )
    return vs[0]


def _body(neural_hbm, feats_hbm, wb_hbm, out_hbm,
          rb0, rb1, rb2, rb3, fb0, fb1, fb2, fb3, ib0, ib1, wbbuf,
          sem_in, sem_fin, sem_out):
    rowbufs = [rb0, rb1, rb2, rb3]
    featbufs = [fb0, fb1, fb2, fb3]
    idxbufs = [ib0, ib1]
    wid = lax.axis_index("s") * NC + lax.axis_index("c")
    base_row = wid * ROWS_PER_W

    pltpu.sync_copy(wb_hbm, wbbuf)
    wv = wbbuf[pl.ds(0, L)]
    w0, w1, w2, b0 = wv[0], wv[1], wv[2], wv[3]

    iota = lax.iota(jnp.int32, L)
    bases = [iota * NSEG + (L * NSEG) * t for t in range(SEG_G)]
    colvecs = [iota + L * g for g in range(SEG_G)]

    def issue_in(b, row):
        return (
            pltpu.async_copy(neural_hbm.at[row], rowbufs[b], sem_in.at[b]),
            pltpu.async_copy(feats_hbm.at[row], featbufs[b], sem_fin.at[b]),
        )

    def wait_in(b, row):
        pltpu.make_async_copy(neural_hbm.at[row], rowbufs[b],
                              sem_in.at[b]).wait()
        pltpu.make_async_copy(feats_hbm.at[row], featbufs[b],
                              sem_fin.at[b]).wait()

    def issue_out(b, row):
        return pltpu.async_copy(rowbufs[b], out_hbm.at[row], sem_out.at[b])

    def wait_out(b, row):
        pltpu.make_async_copy(rowbufs[b], out_hbm.at[row],
                              sem_out.at[b]).wait()

    def compute_pair(bufs, fbufs, ibufs, rows):
        NR = len(bufs)

        def seg_step(v4, MV):
            Ms, Vs = MV
            off = v4 * (NSEG * 4)
            for u in range(4):
                v_id = v4 * 4 + u
                newM, newV = [], []
                for s in range(NR):
                    ms, vs = [], []
                    for g in range(SEG_G):
                        x = bufs[s][pl.ds(off + u * NSEG + g * L, L)]
                        m2 = jnp.maximum(Ms[s][g], x)
                        vs.append(jnp.where(m2 != Ms[s][g], v_id, Vs[s][g]))
                        ms.append(m2)
                    newM.append(tuple(ms))
                    newV.append(tuple(vs))
                Ms, Vs = tuple(newM), tuple(newV)
            return Ms, Vs
        M, V = lax.fori_loop(
            0, SEG_V // 4, seg_step,
            (tuple(tuple(jnp.full((L,), BIG_NEG, jnp.float32)
                         for _ in range(SEG_G)) for _ in range(NR)),
             tuple(tuple(jnp.zeros((L,), jnp.int32)
                         for _ in range(SEG_G)) for _ in range(NR))),
        )

        lane0 = iota == 0

        def extract(k, carry):
            M, V, csum = carry
            out_M, out_V, out_csum = [], [], []
            kvec = jnp.full((L,), 0, jnp.int32) + k
            for s in range(NR):
                buf = bufs[s]
                Ms, Vs = M[s], V[s]
                gmax = _scalar(jnp.max(_maxtree(list(Ms))))
                ecand = [
                    jnp.where(Ms[g] == gmax, Vs[g] * NSEG + colvecs[g],
                              jnp.int32(0x7FFFFFF))
                    for g in range(SEG_G)
                ]
                e = _scalar(jnp.min(_mintree(ecand)))
                plsc.store_scatter(ibufs[s], [kvec], jnp.full((L,), e,
                                                             jnp.int32),
                                   mask=lane0)
                col_e = jnp.remainder(e, NSEG)
                v_e = e // NSEG
                seg = [plsc.load_gather(buf, [bases[t] + col_e])
                       for t in range(SEG_G)]
                lm = iota == jnp.remainder(v_e, L)
                t_e = v_e // L
                segp = [
                    jnp.where(jnp.logical_and(t_e == t, lm),
                              jnp.float32(BIG_NEG), seg[t])
                    for t in range(SEG_G)
                ]
                plsc.store_scatter(buf, [jnp.full((L,), e, jnp.int32)],
                                   jnp.full((L,), BIG_NEG, jnp.float32),
                                   mask=lane0)
                newmax = _scalar(jnp.max(_maxtree(list(segp))))
                vcand = [
                    jnp.where(segp[t] == newmax, iota + (L * t),
                              jnp.int32(99999))
                    for t in range(SEG_G)
                ]
                vnew = _scalar(jnp.min(_mintree(vcand)))
                lane_e = jnp.remainder(col_e, L)
                g_e = col_e // L
                onelane = iota == lane_e
                upd = [jnp.logical_and(g_e == g, onelane)
                       for g in range(SEG_G)]
                out_M.append(tuple(
                    jnp.where(upd[g], newmax, Ms[g]) for g in range(SEG_G)))
                out_V.append(tuple(
                    jnp.where(upd[g], vnew, Vs[g]) for g in range(SEG_G)))
                out_csum.append(csum[s] + gmax)
            return tuple(out_M), tuple(out_V), tuple(out_csum)

        M, V, csum = lax.fori_loop(
            0, K, extract,
            (M, V, tuple(jnp.float32(0.0) for _ in range(NR))))

        for s in range(NR):
            ssum = jnp.zeros((L,), jnp.float32)
            srcs = []
            for j in range(K // L):
                fbase = (iota + j * L) * 3
                f0 = plsc.load_gather(fbufs[s], [fbase])
                f1 = plsc.load_gather(fbufs[s], [fbase + 1])
                f2 = plsc.load_gather(fbufs[s], [fbase + 2])
                sc = jnp.exp(f0 * w0 + f1 * w1 + f2 * w2 + b0)
                srcs.append(sc)
                ssum = ssum + sc
            l1 = jnp.maximum(_scalar(jnp.sum(ssum)), jnp.float32(1e-12))
            scale = jnp.broadcast_to(csum[s], (L,)) / jnp.broadcast_to(l1, (L,))
            for j in range(K // L):
                idx = ibufs[s][pl.ds(j * L, L)]
                plsc.store_scatter(bufs[s], [idx], srcs[j] * scale)

    issue_in(0, base_row + 0)
    issue_in(1, base_row + 1)

    def pipeline_body(i2, _):
        q = base_row + i2 * NBUF

        @pl.when(i2 > 0)
        def _():
            wait_out(2, q - 2)
            wait_out(3, q - 1)

        issue_in(2, q + 2)
        issue_in(3, q + 3)

        wait_in(0, q + 0)
        wait_in(1, q + 1)
        compute_pair([rowbufs[0], rowbufs[1]],
                     [featbufs[0], featbufs[1]], idxbufs, (q, q + 1))
        issue_out(0, q + 0)
        issue_out(1, q + 1)

        wait_in(2, q + 2)
        wait_in(3, q + 3)
        compute_pair([rowbufs[2], rowbufs[3]],
                     [featbufs[2], featbufs[3]], idxbufs, (q + 2, q + 3))
        issue_out(2, q + 2)
        issue_out(3, q + 3)

        wait_out(0, q + 0)
        wait_out(1, q + 1)

        @pl.when(i2 < NBODY - 1)
        def _():
            issue_in(0, q + 4)
            issue_in(1, q + 5)

        return 0

    lax.fori_loop(0, NBODY, pipeline_body, 0)
    last = base_row + (NBODY - 1) * NBUF
    wait_out(2, last + 2)
    wait_out(3, last + 3)


@jax.jit
def kernel(neural_prob_mtx, features, W, b):
    feats = features.reshape(N1, K * 3)
    wb = jnp.zeros((16,), jnp.float32)
    wb = wb.at[0].set(W[0, 0]).at[1].set(W[1, 0]).at[2].set(W[2, 0]).at[3].set(b[0])

    mesh = plsc.VectorSubcoreMesh(core_axis_name="c", subcore_axis_name="s")
    run = pl.kernel(
        _body,
        out_type=jax.ShapeDtypeStruct((N1, N2), jnp.float32),
        mesh=mesh,
        scratch_types=[
            pltpu.VMEM((N2,), jnp.float32),
            pltpu.VMEM((N2,), jnp.float32),
            pltpu.VMEM((N2,), jnp.float32),
            pltpu.VMEM((N2,), jnp.float32),
            pltpu.VMEM((K * 3,), jnp.float32),
            pltpu.VMEM((K * 3,), jnp.float32),
            pltpu.VMEM((K * 3,), jnp.float32),
            pltpu.VMEM((K * 3,), jnp.float32),
            pltpu.VMEM((K,), jnp.int32),
            pltpu.VMEM((K,), jnp.int32),
            pltpu.VMEM((16,), jnp.float32),
            pltpu.SemaphoreType.DMA((NBUF,)),
            pltpu.SemaphoreType.DMA((NBUF,)),
            pltpu.SemaphoreType.DMA((NBUF,)),
        ],
        compiler_params=pltpu.CompilerParams(needs_layout_passes=False),
    )
    return run(neural_prob_mtx, feats, wb)


# speedup vs baseline: 16.0499x; 1.0079x over previous
import jax
import jax.numpy as jnp
from jax import lax
from jax.experimental import pallas as pl
from jax.experimental.pallas import tpu as pltpu
from jax.experimental.pallas import tpu_sc as plsc

N1 = 4096
N2 = 16384
K = 128
NC = 2
NS = 16
L = 16
NW = NC * NS
ROWS_PER_W = N1 // NW
NSEG = 128
SEG_G = 8
SEG_V = N2 // NSEG
BIG_NEG = -3.0e38
NBUF = 4
NBODY = ROWS_PER_W // NBUF


def _scalar(x):
    if getattr(x, "shape", ()) == (L,):
        return x[0]
    return x


def _maxtree(vs):
    while len(vs) > 1:
        vs = [jnp.maximum(vs[2 * i], vs[2 * i + 1]) for i in range(len(vs) // 2)] + (
            [vs[-1]] if len(vs) % 2 else []
        )
    return vs[0]


def _mintree(vs):
    while len(vs) > 1:
        vs = [jnp.minimum(vs[2 * i], vs[2 * i + 1]) for i in range(len(vs) // 2)] + (
            [vs[-1]] if len(vs) % 2 else []
        )
    return vs[0]


def _body(neural_hbm, feats_hbm, wb_hbm, out_hbm,
          rb0, rb1, rb2, rb3, fb0, fb1, fb2, fb3, ib0, ib1, wbbuf,
          sem_in, sem_fin, sem_out):
    rowbufs = [rb0, rb1, rb2, rb3]
    featbufs = [fb0, fb1, fb2, fb3]
    idxbufs = [ib0, ib1]
    wid = lax.axis_index("s") * NC + lax.axis_index("c")
    base_row = wid * ROWS_PER_W

    pltpu.sync_copy(wb_hbm, wbbuf)
    wv = wbbuf[pl.ds(0, L)]
    w0, w1, w2, b0 = wv[0], wv[1], wv[2], wv[3]

    iota = lax.iota(jnp.int32, L)
    bases = [iota * NSEG + (L * NSEG) * t for t in range(SEG_G)]
    colvecs = [iota + L * g for g in range(SEG_G)]

    def issue_in(b, row):
        return (
            pltpu.async_copy(neural_hbm.at[row], rowbufs[b], sem_in.at[b]),
            pltpu.async_copy(feats_hbm.at[row], featbufs[b], sem_fin.at[b]),
        )

    def wait_in(b, row):
        pltpu.make_async_copy(neural_hbm.at[row], rowbufs[b],
                              sem_in.at[b]).wait()
        pltpu.make_async_copy(feats_hbm.at[row], featbufs[b],
                              sem_fin.at[b]).wait()

    def issue_out(b, row):
        return pltpu.async_copy(rowbufs[b], out_hbm.at[row], sem_out.at[b])

    def wait_out(b, row):
        pltpu.make_async_copy(rowbufs[b], out_hbm.at[row],
                              sem_out.at[b]).wait()

    def compute_pair(bufs, fbufs, ibufs, rows):
        NR = len(bufs)

        def seg_step(v4, MV):
            Ms, Vs = MV
            off = v4 * (NSEG * 4)
            for u in range(4):
                eoff = off + u * NSEG
                newM, newV = [], []
                for s in range(NR):
                    ms, vs = [], []
                    for g in range(SEG_G):
                        x = bufs[s][pl.ds(eoff + g * L, L)]
                        m2 = jnp.maximum(Ms[s][g], x)
                        vs.append(jnp.where(m2 != Ms[s][g],
                                            colvecs[g] + eoff, Vs[s][g]))
                        ms.append(m2)
                    newM.append(tuple(ms))
                    newV.append(tuple(vs))
                Ms, Vs = tuple(newM), tuple(newV)
            return Ms, Vs
        M, V = lax.fori_loop(
            0, SEG_V // 4, seg_step,
            (tuple(tuple(jnp.full((L,), BIG_NEG, jnp.float32)
                         for _ in range(SEG_G)) for _ in range(NR)),
             tuple(tuple(jnp.zeros((L,), jnp.int32)
                         for _ in range(SEG_G)) for _ in range(NR))),
        )

        lane0 = iota == 0

        def extract(k, carry):
            M, V, csum = carry
            out_M, out_V, out_csum = [], [], []
            kvec = jnp.full((L,), 0, jnp.int32) + k
            for s in range(NR):
                buf = bufs[s]
                Ms, Vs = M[s], V[s]
                gmax = _scalar(jnp.max(_maxtree(list(Ms))))
                ecand = [
                    jnp.where(Ms[g] == gmax, Vs[g], jnp.int32(0x7FFFFFF))
                    for g in range(SEG_G)
                ]
                e = _scalar(jnp.min(_mintree(ecand)))
                plsc.store_scatter(ibufs[s], [kvec], jnp.full((L,), e,
                                                             jnp.int32),
                                   mask=lane0)
                col_e = jnp.remainder(e, NSEG)
                plsc.store_scatter(buf, [jnp.full((L,), e, jnp.int32)],
                                   jnp.full((L,), BIG_NEG, jnp.float32),
                                   mask=lane0)
                seg = [plsc.load_gather(buf, [bases[t] + col_e])
                       for t in range(SEG_G)]
                newmax = _scalar(jnp.max(_maxtree(list(seg))))
                vnew = jnp.int32(99999)
                for t in range(SEG_G):
                    mt = seg[t] == newmax
                    pc = _scalar(plsc.all_reduce_population_count(mt))
                    ff = _scalar(plsc.all_reduce_ffs(mt))
                    vnew = jnp.minimum(
                        vnew, jnp.where(pc > 0, ff + L * t, jnp.int32(99999)))
                ev_new = vnew * NSEG + col_e
                upd = [colvecs[g] == col_e for g in range(SEG_G)]
                out_M.append(tuple(
                    jnp.where(upd[g], newmax, Ms[g]) for g in range(SEG_G)))
                out_V.append(tuple(
                    jnp.where(upd[g], ev_new, Vs[g]) for g in range(SEG_G)))
                out_csum.append(csum[s] + gmax)
            return tuple(out_M), tuple(out_V), tuple(out_csum)

        M, V, csum = lax.fori_loop(
            0, K, extract,
            (M, V, tuple(jnp.float32(0.0) for _ in range(NR))))

        for s in range(NR):
            ssum = jnp.zeros((L,), jnp.float32)
            srcs = []
            for j in range(K // L):
                fbase = (iota + j * L) * 3
                f0 = plsc.load_gather(fbufs[s], [fbase])
                f1 = plsc.load_gather(fbufs[s], [fbase + 1])
                f2 = plsc.load_gather(fbufs[s], [fbase + 2])
                sc = jnp.exp(f0 * w0 + f1 * w1 + f2 * w2 + b0)
                srcs.append(sc)
                ssum = ssum + sc
            l1 = jnp.maximum(_scalar(jnp.sum(ssum)), jnp.float32(1e-12))
            scale = jnp.broadcast_to(csum[s], (L,)) / jnp.broadcast_to(l1, (L,))
            for j in range(K // L):
                idx = ibufs[s][pl.ds(j * L, L)]
                plsc.store_scatter(bufs[s], [idx], srcs[j] * scale)

    issue_in(0, base_row + 0)
    issue_in(1, base_row + 1)

    def pipeline_body(i2, _):
        q = base_row + i2 * NBUF

        @pl.when(i2 > 0)
        def _():
            wait_out(2, q - 2)
            wait_out(3, q - 1)

        issue_in(2, q + 2)
        issue_in(3, q + 3)

        wait_in(0, q + 0)
        wait_in(1, q + 1)
        compute_pair([rowbufs[0], rowbufs[1]],
                     [featbufs[0], featbufs[1]], idxbufs, (q, q + 1))
        issue_out(0, q + 0)
        issue_out(1, q + 1)

        wait_in(2, q + 2)
        wait_in(3, q + 3)
        compute_pair([rowbufs[2], rowbufs[3]],
                     [featbufs[2], featbufs[3]], idxbufs, (q + 2, q + 3))
        issue_out(2, q + 2)
        issue_out(3, q + 3)

        wait_out(0, q + 0)
        wait_out(1, q + 1)

        @pl.when(i2 < NBODY - 1)
        def _():
            issue_in(0, q + 4)
            issue_in(1, q + 5)

        return 0

    lax.fori_loop(0, NBODY, pipeline_body, 0)
    last = base_row + (NBODY - 1) * NBUF
    wait_out(2, last + 2)
    wait_out(3, last + 3)


@jax.jit
def kernel(neural_prob_mtx, features, W, b):
    feats = features.reshape(N1, K * 3)
    wb = jnp.zeros((16,), jnp.float32)
    wb = wb.at[0].set(W[0, 0]).at[1].set(W[1, 0]).at[2].set(W[2, 0]).at[3].set(b[0])

    mesh = plsc.VectorSubcoreMesh(core_axis_name="c", subcore_axis_name="s")
    run = pl.kernel(
        _body,
        out_type=jax.ShapeDtypeStruct((N1, N2), jnp.float32),
        mesh=mesh,
        scratch_types=[
            pltpu.VMEM((N2,), jnp.float32),
            pltpu.VMEM((N2,), jnp.float32),
            pltpu.VMEM((N2,), jnp.float32),
            pltpu.VMEM((N2,), jnp.float32),
            pltpu.VMEM((K * 3,), jnp.float32),
            pltpu.VMEM((K * 3,), jnp.float32),
            pltpu.VMEM((K * 3,), jnp.float32),
            pltpu.VMEM((K * 3,), jnp.float32),
            pltpu.VMEM((K,), jnp.int32),
            pltpu.VMEM((K,), jnp.int32),
            pltpu.VMEM((16,), jnp.float32),
            pltpu.SemaphoreType.DMA((NBUF,)),
            pltpu.SemaphoreType.DMA((NBUF,)),
            pltpu.SemaphoreType.DMA((NBUF,)),
        ],
        compiler_params=pltpu.CompilerParams(needs_layout_passes=False),
    )
    return run(neural_prob_mtx, feats, wb)


# speedup vs baseline: 18.0664x; 1.1256x over previous
import jax
import jax.numpy as jnp
from jax import lax
from jax.experimental import pallas as pl
from jax.experimental.pallas import tpu as pltpu
from jax.experimental.pallas import tpu_sc as plsc

N1 = 4096
N2 = 16384
K = 128
NC = 2
NS = 16
L = 16
NW = NC * NS
ROWS_PER_W = N1 // NW
NSEG = 128
SEG_G = 8
SEG_V = N2 // NSEG
BIG_NEG = -3.0e38
NBUF = 4
NBODY = ROWS_PER_W // NBUF


def _scalar(x):
    if getattr(x, "shape", ()) == (L,):
        return x[0]
    return x


def _maxtree(vs):
    while len(vs) > 1:
        vs = [jnp.maximum(vs[2 * i], vs[2 * i + 1]) for i in range(len(vs) // 2)] + (
            [vs[-1]] if len(vs) % 2 else []
        )
    return vs[0]


def _mintree(vs):
    while len(vs) > 1:
        vs = [jnp.minimum(vs[2 * i], vs[2 * i + 1]) for i in range(len(vs) // 2)] + (
            [vs[-1]] if len(vs) % 2 else []
        )
    return vs[0]


def _body(neural_hbm, feats_hbm, wb_hbm, out_hbm,
          rb0, rb1, rb2, rb3, fb0, fb1, fb2, fb3, ib0, ib1, wbbuf,
          sem_in, sem_fin, sem_out):
    rowbufs = [rb0, rb1, rb2, rb3]
    featbufs = [fb0, fb1, fb2, fb3]
    idxbufs = [ib0, ib1]
    wid = lax.axis_index("s") * NC + lax.axis_index("c")
    base_row = wid * ROWS_PER_W

    pltpu.sync_copy(wb_hbm, wbbuf)
    wv = wbbuf[pl.ds(0, L)]
    w0, w1, w2, b0 = wv[0], wv[1], wv[2], wv[3]

    iota = lax.iota(jnp.int32, L)
    bases = [iota * NSEG + (L * NSEG) * t for t in range(SEG_G)]
    colvecs = [iota + L * g for g in range(SEG_G)]

    def issue_in(b, row):
        return (
            pltpu.async_copy(neural_hbm.at[row], rowbufs[b], sem_in.at[b]),
            pltpu.async_copy(feats_hbm.at[row], featbufs[b], sem_fin.at[b]),
        )

    def wait_in(b, row):
        pltpu.make_async_copy(neural_hbm.at[row], rowbufs[b],
                              sem_in.at[b]).wait()
        pltpu.make_async_copy(feats_hbm.at[row], featbufs[b],
                              sem_fin.at[b]).wait()

    def issue_out(b, row):
        return pltpu.async_copy(rowbufs[b], out_hbm.at[row], sem_out.at[b])

    def wait_out(b, row):
        pltpu.make_async_copy(rowbufs[b], out_hbm.at[row],
                              sem_out.at[b]).wait()

    def compute_pair(bufs, fbufs, ibufs, rows):
        NR = len(bufs)

        def seg_step(v4, MV):
            Ms, Vs = MV
            off = v4 * (NSEG * 4)
            for u in range(4):
                eoff = off + u * NSEG
                newM, newV = [], []
                for s in range(NR):
                    ms, vs = [], []
                    for g in range(SEG_G):
                        x = bufs[s][pl.ds(eoff + g * L, L)]
                        m2 = jnp.maximum(Ms[s][g], x)
                        vs.append(jnp.where(m2 != Ms[s][g],
                                            colvecs[g] + eoff, Vs[s][g]))
                        ms.append(m2)
                    newM.append(tuple(ms))
                    newV.append(tuple(vs))
                Ms, Vs = tuple(newM), tuple(newV)
            return Ms, Vs
        M, V = lax.fori_loop(
            0, SEG_V // 4, seg_step,
            (tuple(tuple(jnp.full((L,), BIG_NEG, jnp.float32)
                         for _ in range(SEG_G)) for _ in range(NR)),
             tuple(tuple(jnp.zeros((L,), jnp.int32)
                         for _ in range(SEG_G)) for _ in range(NR))),
        )

        lane0 = iota == 0

        def extract(k, carry):
            M, V, csum, gmaxs = carry
            kvec = jnp.full((L,), 0, jnp.int32) + k

            es, cols, upds = [], [], []
            for s in range(NR):
                ecand = [
                    jnp.where(M[s][g] == gmaxs[s], V[s][g],
                              jnp.int32(0x7FFFFFF))
                    for g in range(SEG_G)
                ]
                e = _scalar(jnp.min(_mintree(ecand)))
                es.append(e)
                cols.append(jnp.remainder(e, NSEG))
                upds.append([colvecs[g] == cols[s] for g in range(SEG_G)])

            for s in range(NR):
                plsc.store_scatter(ibufs[s], [kvec],
                                   jnp.full((L,), es[s], jnp.int32),
                                   mask=lane0)
            for s in range(NR):
                plsc.store_scatter(bufs[s], [jnp.full((L,), es[s], jnp.int32)],
                                   jnp.full((L,), BIG_NEG, jnp.float32),
                                   mask=lane0)
            segs = [
                [plsc.load_gather(bufs[s], [bases[t] + cols[s]])
                 for t in range(SEG_G)]
                for s in range(NR)
            ]

            out_M, out_V, out_csum, out_gmax = [], [], [], []
            for s in range(NR):
                seg = segs[s]
                newmax = _scalar(jnp.max(_maxtree(list(seg))))
                vnew = jnp.int32(99999)
                for t in range(SEG_G):
                    mt = seg[t] == newmax
                    pc = _scalar(plsc.all_reduce_population_count(mt))
                    ff = _scalar(plsc.all_reduce_ffs(mt))
                    vnew = jnp.minimum(
                        vnew, jnp.where(pc > 0, ff + L * t, jnp.int32(99999)))
                ev_new = vnew * NSEG + cols[s]
                m2 = _scalar(jnp.max(_maxtree([
                    jnp.where(upds[s][g], jnp.float32(BIG_NEG), M[s][g])
                    for g in range(SEG_G)
                ])))
                out_M.append(tuple(
                    jnp.where(upds[s][g], newmax, M[s][g])
                    for g in range(SEG_G)))
                out_V.append(tuple(
                    jnp.where(upds[s][g], ev_new, V[s][g])
                    for g in range(SEG_G)))
                out_csum.append(csum[s] + gmaxs[s])
                out_gmax.append(jnp.maximum(m2, newmax))
            return (tuple(out_M), tuple(out_V), tuple(out_csum),
                    tuple(out_gmax))

        gmax0 = tuple(_scalar(jnp.max(_maxtree(list(M[s]))))
                      for s in range(NR))
        M, V, csum, _ = lax.fori_loop(
            0, K, extract,
            (M, V, tuple(jnp.float32(0.0) for _ in range(NR)), gmax0))

        for s in range(NR):
            ssum = jnp.zeros((L,), jnp.float32)
            srcs = []
            for j in range(K // L):
                fbase = (iota + j * L) * 3
                f0 = plsc.load_gather(fbufs[s], [fbase])
                f1 = plsc.load_gather(fbufs[s], [fbase + 1])
                f2 = plsc.load_gather(fbufs[s], [fbase + 2])
                sc = jnp.exp(f0 * w0 + f1 * w1 + f2 * w2 + b0)
                srcs.append(sc)
                ssum = ssum + sc
            l1 = jnp.maximum(_scalar(jnp.sum(ssum)), jnp.float32(1e-12))
            scale = jnp.broadcast_to(csum[s], (L,)) / jnp.broadcast_to(l1, (L,))
            for j in range(K // L):
                idx = ibufs[s][pl.ds(j * L, L)]
                plsc.store_scatter(bufs[s], [idx], srcs[j] * scale)

    issue_in(0, base_row + 0)
    issue_in(1, base_row + 1)

    def pipeline_body(i2, _):
        q = base_row + i2 * NBUF

        @pl.when(i2 > 0)
        def _():
            wait_out(2, q - 2)
            wait_out(3, q - 1)

        issue_in(2, q + 2)
        issue_in(3, q + 3)

        wait_in(0, q + 0)
        wait_in(1, q + 1)
        compute_pair([rowbufs[0], rowbufs[1]],
                     [featbufs[0], featbufs[1]], idxbufs, (q, q + 1))
        issue_out(0, q + 0)
        issue_out(1, q + 1)

        wait_in(2, q + 2)
        wait_in(3, q + 3)
        compute_pair([rowbufs[2], rowbufs[3]],
                     [featbufs[2], featbufs[3]], idxbufs, (q + 2, q + 3))
        issue_out(2, q + 2)
        issue_out(3, q + 3)

        wait_out(0, q + 0)
        wait_out(1, q + 1)

        @pl.when(i2 < NBODY - 1)
        def _():
            issue_in(0, q + 4)
            issue_in(1, q + 5)

        return 0

    lax.fori_loop(0, NBODY, pipeline_body, 0)
    last = base_row + (NBODY - 1) * NBUF
    wait_out(2, last + 2)
    wait_out(3, last + 3)


@jax.jit
def kernel(neural_prob_mtx, features, W, b):
    feats = features.reshape(N1, K * 3)
    wb = jnp.zeros((16,), jnp.float32)
    wb = wb.at[0].set(W[0, 0]).at[1].set(W[1, 0]).at[2].set(W[2, 0]).at[3].set(b[0])

    mesh = plsc.VectorSubcoreMesh(core_axis_name="c", subcore_axis_name="s")
    run = pl.kernel(
        _body,
        out_type=jax.ShapeDtypeStruct((N1, N2), jnp.float32),
        mesh=mesh,
        scratch_types=[
            pltpu.VMEM((N2,), jnp.float32),
            pltpu.VMEM((N2,), jnp.float32),
            pltpu.VMEM((N2,), jnp.float32),
            pltpu.VMEM((N2,), jnp.float32),
            pltpu.VMEM((K * 3,), jnp.float32),
            pltpu.VMEM((K * 3,), jnp.float32),
            pltpu.VMEM((K * 3,), jnp.float32),
            pltpu.VMEM((K * 3,), jnp.float32),
            pltpu.VMEM((K,), jnp.int32),
            pltpu.VMEM((K,), jnp.int32),
            pltpu.VMEM((16,), jnp.float32),
            pltpu.SemaphoreType.DMA((NBUF,)),
            pltpu.SemaphoreType.DMA((NBUF,)),
            pltpu.SemaphoreType.DMA((NBUF,)),
        ],
        compiler_params=pltpu.CompilerParams(needs_layout_passes=False),
    )
    return run(neural_prob_mtx, feats, wb)


# speedup vs baseline: 18.4516x; 1.0213x over previous
import jax
import jax.numpy as jnp
from jax import lax
from jax.experimental import pallas as pl
from jax.experimental.pallas import tpu as pltpu
from jax.experimental.pallas import tpu_sc as plsc

N1 = 4096
N2 = 16384
K = 128
NC = 2
NS = 16
L = 16
NW = NC * NS
ROWS_PER_W = N1 // NW
NSEG = 128
SEG_G = 8
SEG_V = N2 // NSEG
BIG_NEG = -3.0e38
NR = 3
NBUF = 2 * NR
NBODY = 21


def _scalar(x):
    if getattr(x, "shape", ()) == (L,):
        return x[0]
    return x


def _maxtree(vs):
    while len(vs) > 1:
        vs = [jnp.maximum(vs[2 * i], vs[2 * i + 1]) for i in range(len(vs) // 2)] + (
            [vs[-1]] if len(vs) % 2 else []
        )
    return vs[0]


def _mintree(vs):
    while len(vs) > 1:
        vs = [jnp.minimum(vs[2 * i], vs[2 * i + 1]) for i in range(len(vs) // 2)] + (
            [vs[-1]] if len(vs) % 2 else []
        )
    return vs[0]


def _body(neural_hbm, feats_hbm, wb_hbm, out_hbm,
          rb0, rb1, rb2, rb3, rb4, rb5, fb0, fb1, fb2, fb3, fb4, fb5,
          ib0, ib1, ib2, wbbuf, sem_in, sem_fin, sem_out):
    rowbufs = [rb0, rb1, rb2, rb3, rb4, rb5]
    featbufs = [fb0, fb1, fb2, fb3, fb4, fb5]
    idxbufs = [ib0, ib1, ib2]
    wid = lax.axis_index("s") * NC + lax.axis_index("c")
    base_row = wid * ROWS_PER_W

    pltpu.sync_copy(wb_hbm, wbbuf)
    wv = wbbuf[pl.ds(0, L)]
    w0, w1, w2, b0 = wv[0], wv[1], wv[2], wv[3]

    iota = lax.iota(jnp.int32, L)
    bases = [iota * NSEG + (L * NSEG) * t for t in range(SEG_G)]
    colvecs = [iota + L * g for g in range(SEG_G)]

    def issue_in(b, row):
        return (
            pltpu.async_copy(neural_hbm.at[row], rowbufs[b], sem_in.at[b]),
            pltpu.async_copy(feats_hbm.at[row], featbufs[b], sem_fin.at[b]),
        )

    def wait_in(b, row):
        pltpu.make_async_copy(neural_hbm.at[row], rowbufs[b],
                              sem_in.at[b]).wait()
        pltpu.make_async_copy(feats_hbm.at[row], featbufs[b],
                              sem_fin.at[b]).wait()

    def issue_out(b, row):
        return pltpu.async_copy(rowbufs[b], out_hbm.at[row], sem_out.at[b])

    def wait_out(b, row):
        pltpu.make_async_copy(rowbufs[b], out_hbm.at[row],
                              sem_out.at[b]).wait()

    def compute_pair(bufs, fbufs, ibufs, rows):
        NR = len(bufs)

        def seg_step(v4, MV):
            Ms, Vs = MV
            off = v4 * (NSEG * 4)
            for u in range(4):
                eoff = off + u * NSEG
                newM, newV = [], []
                for s in range(NR):
                    ms, vs = [], []
                    for g in range(SEG_G):
                        x = bufs[s][pl.ds(eoff + g * L, L)]
                        m2 = jnp.maximum(Ms[s][g], x)
                        vs.append(jnp.where(m2 != Ms[s][g],
                                            colvecs[g] + eoff, Vs[s][g]))
                        ms.append(m2)
                    newM.append(tuple(ms))
                    newV.append(tuple(vs))
                Ms, Vs = tuple(newM), tuple(newV)
            return Ms, Vs
        M, V = lax.fori_loop(
            0, SEG_V // 4, seg_step,
            (tuple(tuple(jnp.full((L,), BIG_NEG, jnp.float32)
                         for _ in range(SEG_G)) for _ in range(NR)),
             tuple(tuple(jnp.zeros((L,), jnp.int32)
                         for _ in range(SEG_G)) for _ in range(NR))),
        )

        lane0 = iota == 0

        def extract(k, carry):
            M, V, csum, gmaxs = carry
            kvec = jnp.full((L,), 0, jnp.int32) + k

            es, cols, upds = [], [], []
            for s in range(NR):
                ecand = [
                    jnp.where(M[s][g] == gmaxs[s], V[s][g],
                              jnp.int32(0x7FFFFFF))
                    for g in range(SEG_G)
                ]
                e = _scalar(jnp.min(_mintree(ecand)))
                es.append(e)
                cols.append(jnp.remainder(e, NSEG))
                upds.append([colvecs[g] == cols[s] for g in range(SEG_G)])

            for s in range(NR):
                plsc.store_scatter(ibufs[s], [kvec],
                                   jnp.full((L,), es[s], jnp.int32),
                                   mask=lane0)
            for s in range(NR):
                plsc.store_scatter(bufs[s], [jnp.full((L,), es[s], jnp.int32)],
                                   jnp.full((L,), BIG_NEG, jnp.float32),
                                   mask=lane0)
            segs = [
                [plsc.load_gather(bufs[s], [bases[t] + cols[s]])
                 for t in range(SEG_G)]
                for s in range(NR)
            ]

            out_M, out_V, out_csum, out_gmax = [], [], [], []
            for s in range(NR):
                seg = segs[s]
                newmax = _scalar(jnp.max(_maxtree(list(seg))))
                vnew = jnp.int32(99999)
                for t in range(SEG_G):
                    mt = seg[t] == newmax
                    pc = _scalar(plsc.all_reduce_population_count(mt))
                    ff = _scalar(plsc.all_reduce_ffs(mt))
                    vnew = jnp.minimum(
                        vnew, jnp.where(pc > 0, ff + L * t, jnp.int32(99999)))
                ev_new = vnew * NSEG + cols[s]
                m2 = _scalar(jnp.max(_maxtree([
                    jnp.where(upds[s][g], jnp.float32(BIG_NEG), M[s][g])
                    for g in range(SEG_G)
                ])))
                out_M.append(tuple(
                    jnp.where(upds[s][g], newmax, M[s][g])
                    for g in range(SEG_G)))
                out_V.append(tuple(
                    jnp.where(upds[s][g], ev_new, V[s][g])
                    for g in range(SEG_G)))
                out_csum.append(csum[s] + gmaxs[s])
                out_gmax.append(jnp.maximum(m2, newmax))
            return (tuple(out_M), tuple(out_V), tuple(out_csum),
                    tuple(out_gmax))

        gmax0 = tuple(_scalar(jnp.max(_maxtree(list(M[s]))))
                      for s in range(NR))
        M, V, csum, _ = lax.fori_loop(
            0, K, extract,
            (M, V, tuple(jnp.float32(0.0) for _ in range(NR)), gmax0))

        for s in range(NR):
            ssum = jnp.zeros((L,), jnp.float32)
            srcs = []
            for j in range(K // L):
                fbase = (iota + j * L) * 3
                f0 = plsc.load_gather(fbufs[s], [fbase])
                f1 = plsc.load_gather(fbufs[s], [fbase + 1])
                f2 = plsc.load_gather(fbufs[s], [fbase + 2])
                sc = jnp.exp(f0 * w0 + f1 * w1 + f2 * w2 + b0)
                srcs.append(sc)
                ssum = ssum + sc
            l1 = jnp.maximum(_scalar(jnp.sum(ssum)), jnp.float32(1e-12))
            scale = jnp.broadcast_to(csum[s], (L,)) / jnp.broadcast_to(l1, (L,))
            for j in range(K // L):
                idx = ibufs[s][pl.ds(j * L, L)]
                plsc.store_scatter(bufs[s], [idx], srcs[j] * scale)

    for b in range(NR):
        issue_in(b, base_row + b)

    def pipeline_body(i2, _):
        q = base_row + i2 * NBUF

        @pl.when(i2 > 0)
        def _():
            for b in range(NR):
                wait_out(NR + b, q - NR + b)

        for b in range(NR):
            issue_in(NR + b, q + NR + b)

        for b in range(NR):
            wait_in(b, q + b)
        compute_pair([rowbufs[b] for b in range(NR)],
                     [featbufs[b] for b in range(NR)], idxbufs,
                     tuple(q + b for b in range(NR)))
        for b in range(NR):
            issue_out(b, q + b)

        for b in range(NR):
            wait_in(NR + b, q + NR + b)
        compute_pair([rowbufs[NR + b] for b in range(NR)],
                     [featbufs[NR + b] for b in range(NR)], idxbufs,
                     tuple(q + NR + b for b in range(NR)))
        for b in range(NR):
            issue_out(NR + b, q + NR + b)

        for b in range(NR):
            wait_out(b, q + b)

        @pl.when(i2 < NBODY - 1)
        def _():
            for b in range(NR):
                issue_in(b, q + NBUF + b)

        return 0

    lax.fori_loop(0, NBODY, pipeline_body, 0)
    last = base_row + (NBODY - 1) * NBUF
    for b in range(NR):
        wait_out(NR + b, last + NR + b)

    tail = base_row + NBODY * NBUF
    issue_in(0, tail)
    issue_in(1, tail + 1)
    wait_in(0, tail)
    wait_in(1, tail + 1)
    compute_pair([rowbufs[0], rowbufs[1]], [featbufs[0], featbufs[1]],
                 idxbufs, (tail, tail + 1))
    issue_out(0, tail)
    issue_out(1, tail + 1)
    wait_out(0, tail)
    wait_out(1, tail + 1)


@jax.jit
def kernel(neural_prob_mtx, features, W, b):
    feats = features.reshape(N1, K * 3)
    wb = jnp.zeros((16,), jnp.float32)
    wb = wb.at[0].set(W[0, 0]).at[1].set(W[1, 0]).at[2].set(W[2, 0]).at[3].set(b[0])

    mesh = plsc.VectorSubcoreMesh(core_axis_name="c", subcore_axis_name="s")
    run = pl.kernel(
        _body,
        out_type=jax.ShapeDtypeStruct((N1, N2), jnp.float32),
        mesh=mesh,
        scratch_types=[pltpu.VMEM((N2,), jnp.float32)] * NBUF
          + [pltpu.VMEM((K * 3,), jnp.float32)] * NBUF
          + [pltpu.VMEM((K,), jnp.int32)] * NR
          + [
            pltpu.VMEM((16,), jnp.float32),
            pltpu.SemaphoreType.DMA((NBUF,)),
            pltpu.SemaphoreType.DMA((NBUF,)),
            pltpu.SemaphoreType.DMA((NBUF,)),
        ],
        compiler_params=pltpu.CompilerParams(needs_layout_passes=False),
    )
    return run(neural_prob_mtx, feats, wb)
